# baseline (speedup 1.0000x reference)
import functools

import jax
import jax.numpy as jnp
from jax import lax
from jax.experimental import pallas as pl
from jax.experimental.pallas import tpu as pltpu
from jax.experimental.pallas import tpu_sc as plsc

NC = 2
NS = 16
NW = NC * NS
EB = 125
NBUF = 4


def _sc_mesh():
    return plsc.VectorSubcoreMesh(
        core_axis_name="c", subcore_axis_name="s", num_cores=NC, num_subcores=NS
    )


def _make_degree_kernel(n, rows_per_w, width):
    rpt = n // NS

    @functools.partial(
        pl.kernel,
        out_type=jax.ShapeDtypeStruct((NC, 2, n, width), jnp.float32),
        mesh=_sc_mesh(),
        compiler_params=pltpu.CompilerParams(use_tc_tiling_on_sc=False),
        scratch_types=[
            pltpu.VMEM((rows_per_w, EB), jnp.int32),
            pltpu.VMEM((rows_per_w, EB), jnp.int32),
            pltpu.VMEM((EB, width), jnp.float32),
            pltpu.VMEM_SHARED((n, width), jnp.float32),
            pltpu.VMEM_SHARED((n, width), jnp.float32),
            pltpu.SemaphoreType.DMA,
        ],
    )
    def deg_kernel(ei_hbm, ones_hbm, zeros_hbm, out_hbm, sidx, didx, ones_v,
                   table_a, table_b, sem):
        c = lax.axis_index("c")
        s = lax.axis_index("s")
        w = s * NC + c
        pltpu.sync_copy(zeros_hbm, table_a.at[pl.ds(s * rpt, rpt)])
        pltpu.sync_copy(zeros_hbm, table_b.at[pl.ds(s * rpt, rpt)])
        pltpu.sync_copy(ei_hbm.at[0, w], sidx)
        pltpu.sync_copy(ei_hbm.at[1, w], didx)
        pltpu.sync_copy(ones_hbm, ones_v)
        plsc.subcore_barrier()

        def body(j, _):
            pltpu.async_copy(ones_v, table_a.at[sidx.at[j]], sem, add=True)
            pltpu.async_copy(ones_v, table_b.at[didx.at[j]], sem, add=True)

            @pl.when(j >= 2)
            def _():
                pltpu.make_async_copy(ones_v, table_a.at[sidx.at[j]], sem).wait()
                pltpu.make_async_copy(ones_v, table_a.at[sidx.at[j]], sem).wait()

            return ()

        lax.fori_loop(0, rows_per_w, body, ())
        for _ in range(4):
            pltpu.make_async_copy(ones_v, table_a.at[sidx.at[0]], sem).wait()
        plsc.subcore_barrier()
        pltpu.sync_copy(
            table_a.at[pl.ds(s * rpt, rpt)],
            out_hbm.at[c, 0, pl.ds(s * rpt, rpt)],
        )
        pltpu.sync_copy(
            table_b.at[pl.ds(s * rpt, rpt)],
            out_hbm.at[c, 1, pl.ds(s * rpt, rpt)],
        )

    return deg_kernel


def _make_agg_kernel(n, feat, rows_per_w):
    rpt = n // NS

    @functools.partial(
        pl.kernel,
        out_type=jax.ShapeDtypeStruct((NC, n, feat), jnp.float32),
        mesh=_sc_mesh(),
        compiler_params=pltpu.CompilerParams(use_tc_tiling_on_sc=False),
        scratch_types=[
            pltpu.VMEM((rows_per_w, EB), jnp.int32),
            pltpu.VMEM((rows_per_w, EB), jnp.int32),
            pltpu.VMEM((NBUF, EB, feat), jnp.float32),
            pltpu.VMEM_SHARED((n, feat), jnp.float32),
            pltpu.SemaphoreType.DMA((NBUF,)),
            pltpu.SemaphoreType.DMA((NBUF,)),
        ],
    )
    def agg_kernel(h_hbm, ei_hbm, zeros_hbm, out_hbm, sidx, didx, rows,
                   table, gsems, ssems):
        c = lax.axis_index("c")
        s = lax.axis_index("s")
        w = s * NC + c
        pltpu.sync_copy(zeros_hbm, table.at[pl.ds(s * rpt, rpt)])
        pltpu.sync_copy(ei_hbm.at[0, w], sidx)
        pltpu.sync_copy(ei_hbm.at[1, w], didx)
        plsc.subcore_barrier()

        for k in range(NBUF - 1):
            pltpu.async_copy(h_hbm.at[sidx.at[k]], rows.at[k], gsems.at[k])

        def body(j, _):
            slot = lax.rem(j, NBUF)
            nxt = j + NBUF - 1

            @pl.when(nxt < rows_per_w)
            def _():
                nslot = lax.rem(nxt, NBUF)

                @pl.when(j >= 1)
                def _():
                    pltpu.make_async_copy(
                        rows.at[nslot], table.at[didx.at[j]],
                        ssems.at[nslot]).wait()

                pltpu.async_copy(
                    h_hbm.at[sidx.at[nxt]], rows.at[nslot], gsems.at[nslot])

            pltpu.make_async_copy(
                h_hbm.at[sidx.at[j]], rows.at[slot], gsems.at[slot]).wait()
            pltpu.async_copy(
                rows.at[slot], table.at[didx.at[j]], ssems.at[slot], add=True)
            return ()

        lax.fori_loop(0, rows_per_w, body, ())
        for k in range(NBUF):
            pltpu.make_async_copy(
                rows.at[k], table.at[didx.at[0]], ssems.at[k]).wait()
        plsc.subcore_barrier()
        pltpu.sync_copy(
            table.at[pl.ds(s * rpt, rpt)],
            out_hbm.at[c, pl.ds(s * rpt, rpt)],
        )

    return agg_kernel


def _m1_body(x_ref, w0_ref, normo_ref, o_ref):
    h = jnp.dot(x_ref[...], w0_ref[...], preferred_element_type=jnp.float32)
    o_ref[...] = h * normo_ref[...]


def _m2_body(aggp_ref, normi_ref, normo_ref, b0_ref, o_ref):
    agg = aggp_ref[0] + aggp_ref[1]
    hidden = jnp.maximum(agg * normi_ref[...] + b0_ref[...], 0.0)
    o_ref[...] = hidden * normo_ref[...]


def _m3_body(aggp_ref, normi_ref, noise_ref, w1_ref, b1_ref, w2_ref, b2_ref, o_ref):
    t = (aggp_ref[0] + aggp_ref[1]) * normi_ref[...]
    mean = jnp.dot(t, w1_ref[...], preferred_element_type=jnp.float32) + b1_ref[...]
    logstd = jnp.dot(t, w2_ref[...], preferred_element_type=jnp.float32) + b2_ref[...]
    o_ref[...] = noise_ref[...] * jnp.exp(logstd) + mean


def _m4_body(za_ref, zb_ref, o_ref):
    prod = lax.dot_general(
        za_ref[...], zb_ref[...], (((1,), (1,)), ((), ())),
        preferred_element_type=jnp.float32,
    )
    o_ref[...] = 0.5 * jnp.tanh(0.5 * prod) + 0.5


def kernel(x, edge_index, noise, W0, b0, W1, b1, W2, b2):
    n, d = x.shape
    e = edge_index.shape[1]
    h1 = W0.shape[1]
    h2 = W1.shape[1]
    rb = 1000
    gi = n // rb
    dw = 16
    npad = ((n + NS * 8 - 1) // (NS * 8)) * (NS * 8)

    epw = e // NW
    ei4 = edge_index.reshape(2, NW, epw // EB, EB)

    ones_b = jnp.ones((EB, dw), jnp.float32)
    zeros_deg = jnp.zeros((npad // NS, dw), jnp.float32)
    zeros_agg = jnp.zeros((npad // NS, h1), jnp.float32)

    deg_k = _make_degree_kernel(npad, epw // EB, dw)
    degp = deg_k(ei4, ones_b, zeros_deg)
    deg_o = degp[0, 0, :, 0] + degp[1, 0, :, 0]
    deg_i = degp[0, 1, :, 0] + degp[1, 1, :, 0]
    normo = lax.rsqrt(jnp.clip(deg_o, 1.0, None))[:, None]
    normi = lax.rsqrt(jnp.clip(deg_i, 1.0, None))[:, None]

    b0r = b0.reshape(1, h1)
    b1r = b1.reshape(1, h2)
    b2r = b2.reshape(1, h2)

    h_scaled = pl.pallas_call(
        _m1_body,
        grid=(gi,),
        in_specs=[
            pl.BlockSpec((rb, d), lambda i: (i, 0)),
            pl.BlockSpec((d, h1), lambda i: (0, 0)),
            pl.BlockSpec((rb, 1), lambda i: (i, 0)),
        ],
        out_specs=pl.BlockSpec((rb, h1), lambda i: (i, 0)),
        out_shape=jax.ShapeDtypeStruct((n, h1), jnp.float32),
    )(x, W0, normo)

    agg_k = _make_agg_kernel(npad, h1, epw // EB)
    agg1p = agg_k(h_scaled, ei4, zeros_agg)

    hidden_scaled = pl.pallas_call(
        _m2_body,
        grid=(gi,),
        in_specs=[
            pl.BlockSpec((NC, rb, h1), lambda i: (0, i, 0)),
            pl.BlockSpec((rb, 1), lambda i: (i, 0)),
            pl.BlockSpec((rb, 1), lambda i: (i, 0)),
            pl.BlockSpec((1, h1), lambda i: (0, 0)),
        ],
        out_specs=pl.BlockSpec((rb, h1), lambda i: (i, 0)),
        out_shape=jax.ShapeDtypeStruct((n, h1), jnp.float32),
    )(agg1p, normi, normo, b0r)

    agg2p = agg_k(hidden_scaled, ei4, zeros_agg)

    z = pl.pallas_call(
        _m3_body,
        grid=(gi,),
        in_specs=[
            pl.BlockSpec((NC, rb, h1), lambda i: (0, i, 0)),
            pl.BlockSpec((rb, 1), lambda i: (i, 0)),
            pl.BlockSpec((rb, h2), lambda i: (i, 0)),
            pl.BlockSpec((h1, h2), lambda i: (0, 0)),
            pl.BlockSpec((1, h2), lambda i: (0, 0)),
            pl.BlockSpec((h1, h2), lambda i: (0, 0)),
            pl.BlockSpec((1, h2), lambda i: (0, 0)),
        ],
        out_specs=pl.BlockSpec((rb, h2), lambda i: (i, 0)),
        out_shape=jax.ShapeDtypeStruct((n, h2), jnp.float32),
    )(agg2p, normi, noise, W1, b1r, W2, b2r)

    rdec = 400
    a_pred = pl.pallas_call(
        _m4_body,
        grid=(n // rdec,),
        in_specs=[
            pl.BlockSpec((rdec, h2), lambda i: (i, 0)),
            pl.BlockSpec((n, h2), lambda i: (0, 0)),
        ],
        out_specs=pl.BlockSpec((rdec, n), lambda i: (i, 0)),
        out_shape=jax.ShapeDtypeStruct((n, n), jnp.float32),
    )(z, z)

    return (z, a_pred)


# speedup vs baseline: 1.0161x; 1.0161x over previous
import functools

import jax
import jax.numpy as jnp
from jax import lax
from jax.experimental import pallas as pl
from jax.experimental.pallas import tpu as pltpu
from jax.experimental.pallas import tpu_sc as plsc

NC = 2
NS = 16
NW = NC * NS
EB = 125
NBUF = 4


def _sc_mesh():
    return plsc.VectorSubcoreMesh(
        core_axis_name="c", subcore_axis_name="s", num_cores=NC, num_subcores=NS
    )


def _make_degree_kernel(n, rows_per_w, width):
    rpt = n // NS

    @functools.partial(
        pl.kernel,
        out_type=jax.ShapeDtypeStruct((NC, 2, n, width), jnp.float32),
        mesh=_sc_mesh(),
        compiler_params=pltpu.CompilerParams(use_tc_tiling_on_sc=False),
        scratch_types=[
            pltpu.VMEM((rows_per_w, EB), jnp.int32),
            pltpu.VMEM((rows_per_w, EB), jnp.int32),
            pltpu.VMEM((EB, width), jnp.float32),
            pltpu.VMEM_SHARED((n, width), jnp.float32),
            pltpu.VMEM_SHARED((n, width), jnp.float32),
            pltpu.SemaphoreType.DMA,
        ],
    )
    def deg_kernel(ei_hbm, ones_hbm, zeros_hbm, out_hbm, sidx, didx, ones_v,
                   table_a, table_b, sem):
        c = lax.axis_index("c")
        s = lax.axis_index("s")
        w = s * NC + c
        pltpu.sync_copy(zeros_hbm, table_a.at[pl.ds(s * rpt, rpt)])
        pltpu.sync_copy(zeros_hbm, table_b.at[pl.ds(s * rpt, rpt)])
        pltpu.sync_copy(ei_hbm.at[0, w], sidx)
        pltpu.sync_copy(ei_hbm.at[1, w], didx)
        pltpu.sync_copy(ones_hbm, ones_v)
        plsc.subcore_barrier()

        def body(j, _):
            pltpu.async_copy(ones_v, table_a.at[sidx.at[j]], sem, add=True)
            pltpu.async_copy(ones_v, table_b.at[didx.at[j]], sem, add=True)

            @pl.when(j >= 2)
            def _():
                pltpu.make_async_copy(ones_v, table_a.at[sidx.at[j]], sem).wait()
                pltpu.make_async_copy(ones_v, table_a.at[sidx.at[j]], sem).wait()

            return ()

        lax.fori_loop(0, rows_per_w, body, ())
        for _ in range(4):
            pltpu.make_async_copy(ones_v, table_a.at[sidx.at[0]], sem).wait()
        plsc.subcore_barrier()
        pltpu.sync_copy(
            table_a.at[pl.ds(s * rpt, rpt)],
            out_hbm.at[c, 0, pl.ds(s * rpt, rpt)],
        )
        pltpu.sync_copy(
            table_b.at[pl.ds(s * rpt, rpt)],
            out_hbm.at[c, 1, pl.ds(s * rpt, rpt)],
        )

    return deg_kernel


def _make_agg_kernel(n, feat, rows_per_w):
    rpt = n // NS

    @functools.partial(
        pl.kernel,
        out_type=jax.ShapeDtypeStruct((NC, n, feat), jnp.float32),
        mesh=_sc_mesh(),
        compiler_params=pltpu.CompilerParams(use_tc_tiling_on_sc=False),
        scratch_types=[
            pltpu.VMEM((rows_per_w, EB), jnp.int32),
            pltpu.VMEM((rows_per_w, EB), jnp.int32),
            pltpu.VMEM((NBUF, EB, feat), jnp.float32),
            pltpu.VMEM_SHARED((n, feat), jnp.float32),
            pltpu.SemaphoreType.DMA((NBUF,)),
            pltpu.SemaphoreType.DMA((NBUF,)),
        ],
    )
    def agg_kernel(h_hbm, ei_hbm, zeros_hbm, out_hbm, sidx, didx, rows,
                   table, gsems, ssems):
        c = lax.axis_index("c")
        s = lax.axis_index("s")
        w = s * NC + c
        pltpu.sync_copy(zeros_hbm, table.at[pl.ds(s * rpt, rpt)])
        pltpu.sync_copy(ei_hbm.at[0, w], sidx)
        pltpu.sync_copy(ei_hbm.at[1, w], didx)
        plsc.subcore_barrier()

        for k in range(NBUF - 1):
            pltpu.async_copy(h_hbm.at[sidx.at[k]], rows.at[k], gsems.at[k])

        def body(j, _):
            slot = lax.rem(j, NBUF)
            nxt = j + NBUF - 1

            @pl.when(nxt < rows_per_w)
            def _():
                nslot = lax.rem(nxt, NBUF)

                @pl.when(j >= 1)
                def _():
                    pltpu.make_async_copy(
                        rows.at[nslot], table.at[didx.at[j]],
                        ssems.at[nslot]).wait()

                pltpu.async_copy(
                    h_hbm.at[sidx.at[nxt]], rows.at[nslot], gsems.at[nslot])

            pltpu.make_async_copy(
                h_hbm.at[sidx.at[j]], rows.at[slot], gsems.at[slot]).wait()
            pltpu.async_copy(
                rows.at[slot], table.at[didx.at[j]], ssems.at[slot], add=True)
            return ()

        lax.fori_loop(0, rows_per_w, body, ())
        for k in range(NBUF):
            pltpu.make_async_copy(
                rows.at[k], table.at[didx.at[0]], ssems.at[k]).wait()
        plsc.subcore_barrier()
        pltpu.sync_copy(
            table.at[pl.ds(s * rpt, rpt)],
            out_hbm.at[c, pl.ds(s * rpt, rpt)],
        )

    return agg_kernel


def _norm_from(degp_blk):
    deg = degp_blk[0, 0, :, 0:1] + degp_blk[1, 0, :, 0:1]
    return lax.rsqrt(jnp.clip(deg, 1.0, None))


def _m1_body(x_ref, w0_ref, degp_ref, o_ref):
    h = jnp.dot(x_ref[...], w0_ref[...], preferred_element_type=jnp.float32)
    o_ref[...] = h * _norm_from(degp_ref[...])


def _m2_body(aggp_ref, degin_ref, degout_ref, b0_ref, o_ref):
    agg = aggp_ref[0] + aggp_ref[1]
    hidden = jnp.maximum(agg * _norm_from(degin_ref[...]) + b0_ref[...], 0.0)
    o_ref[...] = hidden * _norm_from(degout_ref[...])


def _m3_body(aggp_ref, degin_ref, noise_ref, w1_ref, b1_ref, w2_ref, b2_ref, o_ref):
    t = (aggp_ref[0] + aggp_ref[1]) * _norm_from(degin_ref[...])
    mean = jnp.dot(t, w1_ref[...], preferred_element_type=jnp.float32) + b1_ref[...]
    logstd = jnp.dot(t, w2_ref[...], preferred_element_type=jnp.float32) + b2_ref[...]
    o_ref[...] = noise_ref[...] * jnp.exp(logstd) + mean


def _m4_body(za_ref, zb_ref, o_ref):
    prod = lax.dot_general(
        za_ref[...], zb_ref[...], (((1,), (1,)), ((), ())),
        preferred_element_type=jnp.float32,
    )
    o_ref[...] = 0.5 * jnp.tanh(0.5 * prod) + 0.5


def kernel(x, edge_index, noise, W0, b0, W1, b1, W2, b2):
    n, d = x.shape
    e = edge_index.shape[1]
    h1 = W0.shape[1]
    h2 = W1.shape[1]
    rb = 1000
    gi = n // rb
    dw = 16
    npad = ((n + NS * 8 - 1) // (NS * 8)) * (NS * 8)

    epw = e // NW
    ei4 = edge_index.reshape(2, NW, epw // EB, EB)

    ones_b = jnp.ones((EB, dw), jnp.float32)
    zeros_deg = jnp.zeros((npad // NS, dw), jnp.float32)
    zeros_agg = jnp.zeros((npad // NS, h1), jnp.float32)

    deg_k = _make_degree_kernel(npad, epw // EB, dw)
    degp = deg_k(ei4, ones_b, zeros_deg)

    b0r = b0.reshape(1, h1)
    b1r = b1.reshape(1, h2)
    b2r = b2.reshape(1, h2)

    h_scaled = pl.pallas_call(
        _m1_body,
        grid=(gi,),
        in_specs=[
            pl.BlockSpec((rb, d), lambda i: (i, 0)),
            pl.BlockSpec((d, h1), lambda i: (0, 0)),
            pl.BlockSpec((NC, 1, rb, dw), lambda i: (0, 0, i, 0)),
        ],
        out_specs=pl.BlockSpec((rb, h1), lambda i: (i, 0)),
        out_shape=jax.ShapeDtypeStruct((n, h1), jnp.float32),
    )(x, W0, degp)

    agg_k = _make_agg_kernel(npad, h1, epw // EB)
    agg1p = agg_k(h_scaled, ei4, zeros_agg)

    hidden_scaled = pl.pallas_call(
        _m2_body,
        grid=(gi,),
        in_specs=[
            pl.BlockSpec((NC, rb, h1), lambda i: (0, i, 0)),
            pl.BlockSpec((NC, 1, rb, dw), lambda i: (0, 1, i, 0)),
            pl.BlockSpec((NC, 1, rb, dw), lambda i: (0, 0, i, 0)),
            pl.BlockSpec((1, h1), lambda i: (0, 0)),
        ],
        out_specs=pl.BlockSpec((rb, h1), lambda i: (i, 0)),
        out_shape=jax.ShapeDtypeStruct((n, h1), jnp.float32),
    )(agg1p, degp, degp, b0r)

    agg2p = agg_k(hidden_scaled, ei4, zeros_agg)

    z = pl.pallas_call(
        _m3_body,
        grid=(gi,),
        in_specs=[
            pl.BlockSpec((NC, rb, h1), lambda i: (0, i, 0)),
            pl.BlockSpec((NC, 1, rb, dw), lambda i: (0, 1, i, 0)),
            pl.BlockSpec((rb, h2), lambda i: (i, 0)),
            pl.BlockSpec((h1, h2), lambda i: (0, 0)),
            pl.BlockSpec((1, h2), lambda i: (0, 0)),
            pl.BlockSpec((h1, h2), lambda i: (0, 0)),
            pl.BlockSpec((1, h2), lambda i: (0, 0)),
        ],
        out_specs=pl.BlockSpec((rb, h2), lambda i: (i, 0)),
        out_shape=jax.ShapeDtypeStruct((n, h2), jnp.float32),
    )(agg2p, degp, noise, W1, b1r, W2, b2r)

    rdec = 400
    a_pred = pl.pallas_call(
        _m4_body,
        grid=(n // rdec,),
        in_specs=[
            pl.BlockSpec((rdec, h2), lambda i: (i, 0)),
            pl.BlockSpec((n, h2), lambda i: (0, 0)),
        ],
        out_specs=pl.BlockSpec((rdec, n), lambda i: (i, 0)),
        out_shape=jax.ShapeDtypeStruct((n, n), jnp.float32),
    )(z, z)

    return (z, a_pred)


# speedup vs baseline: 1.0365x; 1.0200x over previous
import functools

import jax
import jax.numpy as jnp
from jax import lax
from jax.experimental import pallas as pl
from jax.experimental.pallas import tpu as pltpu
from jax.experimental.pallas import tpu_sc as plsc

NC = 2
NS = 16
NW = NC * NS
EB = 125
NBUF = 4


def _sc_mesh():
    return plsc.VectorSubcoreMesh(
        core_axis_name="c", subcore_axis_name="s", num_cores=NC, num_subcores=NS
    )


def _make_degree_kernel(n, rows_per_w, width):
    rpt = n // NS

    @functools.partial(
        pl.kernel,
        out_type=jax.ShapeDtypeStruct((NC, 2, n, width), jnp.float32),
        mesh=_sc_mesh(),
        compiler_params=pltpu.CompilerParams(use_tc_tiling_on_sc=False),
        scratch_types=[
            pltpu.VMEM((rows_per_w, EB), jnp.int32),
            pltpu.VMEM((rows_per_w, EB), jnp.int32),
            pltpu.VMEM((EB, width), jnp.float32),
            pltpu.VMEM_SHARED((n, width), jnp.float32),
            pltpu.VMEM_SHARED((n, width), jnp.float32),
            pltpu.SemaphoreType.DMA,
        ],
    )
    def deg_kernel(ei_hbm, ones_hbm, zeros_hbm, out_hbm, sidx, didx, ones_v,
                   table_a, table_b, sem):
        c = lax.axis_index("c")
        s = lax.axis_index("s")
        w = s * NC + c
        pltpu.sync_copy(zeros_hbm, table_a.at[pl.ds(s * rpt, rpt)])
        pltpu.sync_copy(zeros_hbm, table_b.at[pl.ds(s * rpt, rpt)])
        pltpu.sync_copy(ei_hbm.at[0, w], sidx)
        pltpu.sync_copy(ei_hbm.at[1, w], didx)
        pltpu.sync_copy(ones_hbm, ones_v)
        plsc.subcore_barrier()

        def body(j, _):
            pltpu.async_copy(ones_v, table_a.at[sidx.at[j]], sem, add=True)
            pltpu.async_copy(ones_v, table_b.at[didx.at[j]], sem, add=True)

            @pl.when(j >= 2)
            def _():
                pltpu.make_async_copy(ones_v, table_a.at[sidx.at[j]], sem).wait()
                pltpu.make_async_copy(ones_v, table_a.at[sidx.at[j]], sem).wait()

            return ()

        lax.fori_loop(0, rows_per_w, body, ())
        for _ in range(4):
            pltpu.make_async_copy(ones_v, table_a.at[sidx.at[0]], sem).wait()
        plsc.subcore_barrier()
        pltpu.sync_copy(
            table_a.at[pl.ds(s * rpt, rpt)],
            out_hbm.at[c, 0, pl.ds(s * rpt, rpt)],
        )
        pltpu.sync_copy(
            table_b.at[pl.ds(s * rpt, rpt)],
            out_hbm.at[c, 1, pl.ds(s * rpt, rpt)],
        )

    return deg_kernel


def _make_agg_kernel(n, feat, rows_per_w):
    rpt = n // NS

    @functools.partial(
        pl.kernel,
        out_type=jax.ShapeDtypeStruct((NC, n, feat), jnp.float32),
        mesh=_sc_mesh(),
        compiler_params=pltpu.CompilerParams(use_tc_tiling_on_sc=False),
        scratch_types=[
            pltpu.VMEM((rows_per_w, EB), jnp.int32),
            pltpu.VMEM((rows_per_w, EB), jnp.int32),
            pltpu.VMEM((NBUF, EB, feat), jnp.float32),
            pltpu.VMEM_SHARED((n, feat), jnp.float32),
            pltpu.SemaphoreType.DMA((NBUF,)),
            pltpu.SemaphoreType.DMA((NBUF,)),
        ],
    )
    def agg_kernel(h_hbm, ei_hbm, zeros_hbm, out_hbm, sidx, didx, rows,
                   table, gsems, ssems):
        c = lax.axis_index("c")
        s = lax.axis_index("s")
        w = s * NC + c
        pltpu.sync_copy(zeros_hbm, table.at[pl.ds(s * rpt, rpt)])
        pltpu.sync_copy(ei_hbm.at[0, w], sidx)
        pltpu.sync_copy(ei_hbm.at[1, w], didx)
        plsc.subcore_barrier()

        for k in range(NBUF - 1):
            pltpu.async_copy(h_hbm.at[sidx.at[k]], rows.at[k], gsems.at[k])

        def body(j, _):
            slot = lax.rem(j, NBUF)
            nxt = j + NBUF - 1

            @pl.when(nxt < rows_per_w)
            def _():
                nslot = lax.rem(nxt, NBUF)

                @pl.when(j >= 1)
                def _():
                    pltpu.make_async_copy(
                        rows.at[nslot], table.at[didx.at[j]],
                        ssems.at[nslot]).wait()

                pltpu.async_copy(
                    h_hbm.at[sidx.at[nxt]], rows.at[nslot], gsems.at[nslot])

            pltpu.make_async_copy(
                h_hbm.at[sidx.at[j]], rows.at[slot], gsems.at[slot]).wait()
            pltpu.async_copy(
                rows.at[slot], table.at[didx.at[j]], ssems.at[slot], add=True)
            return ()

        lax.fori_loop(0, rows_per_w, body, ())
        for k in range(NBUF):
            pltpu.make_async_copy(
                rows.at[k], table.at[didx.at[0]], ssems.at[k]).wait()
        plsc.subcore_barrier()
        pltpu.sync_copy(
            table.at[pl.ds(s * rpt, rpt)],
            out_hbm.at[c, pl.ds(s * rpt, rpt)],
        )

    return agg_kernel


def _norm_from(degp_blk):
    deg = degp_blk[0, 0, :, 0:1] + degp_blk[1, 0, :, 0:1]
    return lax.rsqrt(jnp.clip(deg, 1.0, None))


def _m1_body(x_ref, w0_ref, degp_ref, o_ref):
    h = jnp.dot(x_ref[...], w0_ref[...], preferred_element_type=jnp.float32)
    o_ref[...] = h * _norm_from(degp_ref[...])


def _m2_body(aggp_ref, degin_ref, degout_ref, b0_ref, o_ref):
    agg = aggp_ref[0] + aggp_ref[1]
    hidden = jnp.maximum(agg * _norm_from(degin_ref[...]) + b0_ref[...], 0.0)
    o_ref[...] = hidden * _norm_from(degout_ref[...])


def _m3_body(aggp_ref, degin_ref, noise_ref, w1_ref, b1_ref, w2_ref, b2_ref, o_ref):
    t = (aggp_ref[0] + aggp_ref[1]) * _norm_from(degin_ref[...])
    mean = jnp.dot(t, w1_ref[...], preferred_element_type=jnp.float32) + b1_ref[...]
    logstd = jnp.dot(t, w2_ref[...], preferred_element_type=jnp.float32) + b2_ref[...]
    o_ref[...] = noise_ref[...] * jnp.exp(logstd) + mean


def _m4_body(za_ref, zb_ref, o_ref):
    prod = lax.dot_general(
        za_ref[...], zb_ref[...], (((1,), (1,)), ((), ())),
        preferred_element_type=jnp.float32,
    )
    o_ref[...] = 0.5 * jnp.tanh(0.5 * prod) + 0.5


def kernel(x, edge_index, noise, W0, b0, W1, b1, W2, b2):
    n, d = x.shape
    e = edge_index.shape[1]
    h1 = W0.shape[1]
    h2 = W1.shape[1]
    rb = 2000
    gi = n // rb
    dw = 16
    npad = ((n + NS * 8 - 1) // (NS * 8)) * (NS * 8)

    epw = e // NW
    ei4 = edge_index.reshape(2, NW, epw // EB, EB)

    ones_b = jnp.ones((EB, dw), jnp.float32)
    zeros_deg = jnp.zeros((npad // NS, dw), jnp.float32)
    zeros_agg = jnp.zeros((npad // NS, h1), jnp.float32)

    deg_k = _make_degree_kernel(npad, epw // EB, dw)
    degp = deg_k(ei4, ones_b, zeros_deg)

    b0r = b0.reshape(1, h1)
    b1r = b1.reshape(1, h2)
    b2r = b2.reshape(1, h2)

    h_scaled = pl.pallas_call(
        _m1_body,
        grid=(gi,),
        in_specs=[
            pl.BlockSpec((rb, d), lambda i: (i, 0)),
            pl.BlockSpec((d, h1), lambda i: (0, 0)),
            pl.BlockSpec((NC, 1, rb, dw), lambda i: (0, 0, i, 0)),
        ],
        out_specs=pl.BlockSpec((rb, h1), lambda i: (i, 0)),
        out_shape=jax.ShapeDtypeStruct((n, h1), jnp.float32),
    )(x, W0, degp)

    agg_k = _make_agg_kernel(npad, h1, epw // EB)
    agg1p = agg_k(h_scaled, ei4, zeros_agg)

    hidden_scaled = pl.pallas_call(
        _m2_body,
        grid=(gi,),
        in_specs=[
            pl.BlockSpec((NC, rb, h1), lambda i: (0, i, 0)),
            pl.BlockSpec((NC, 1, rb, dw), lambda i: (0, 1, i, 0)),
            pl.BlockSpec((NC, 1, rb, dw), lambda i: (0, 0, i, 0)),
            pl.BlockSpec((1, h1), lambda i: (0, 0)),
        ],
        out_specs=pl.BlockSpec((rb, h1), lambda i: (i, 0)),
        out_shape=jax.ShapeDtypeStruct((n, h1), jnp.float32),
    )(agg1p, degp, degp, b0r)

    agg2p = agg_k(hidden_scaled, ei4, zeros_agg)

    z = pl.pallas_call(
        _m3_body,
        grid=(gi,),
        in_specs=[
            pl.BlockSpec((NC, rb, h1), lambda i: (0, i, 0)),
            pl.BlockSpec((NC, 1, rb, dw), lambda i: (0, 1, i, 0)),
            pl.BlockSpec((rb, h2), lambda i: (i, 0)),
            pl.BlockSpec((h1, h2), lambda i: (0, 0)),
            pl.BlockSpec((1, h2), lambda i: (0, 0)),
            pl.BlockSpec((h1, h2), lambda i: (0, 0)),
            pl.BlockSpec((1, h2), lambda i: (0, 0)),
        ],
        out_specs=pl.BlockSpec((rb, h2), lambda i: (i, 0)),
        out_shape=jax.ShapeDtypeStruct((n, h2), jnp.float32),
    )(agg2p, degp, noise, W1, b1r, W2, b2r)

    rdec = 400
    a_pred = pl.pallas_call(
        _m4_body,
        grid=(n // rdec,),
        in_specs=[
            pl.BlockSpec((rdec, h2), lambda i: (i, 0)),
            pl.BlockSpec((n, h2), lambda i: (0, 0)),
        ],
        out_specs=pl.BlockSpec((rdec, n), lambda i: (i, 0)),
        out_shape=jax.ShapeDtypeStruct((n, n), jnp.float32),
    )(z, z)

    return (z, a_pred)


# speedup vs baseline: 1.0439x; 1.0071x over previous
import functools

import jax
import jax.numpy as jnp
from jax import lax
from jax.experimental import pallas as pl
from jax.experimental.pallas import tpu as pltpu
from jax.experimental.pallas import tpu_sc as plsc

NC = 2
NS = 16
NW = NC * NS
EB = 125
NBUF = 4


def _sc_mesh():
    return plsc.VectorSubcoreMesh(
        core_axis_name="c", subcore_axis_name="s", num_cores=NC, num_subcores=NS
    )


def _make_degree_kernel(n, rows_per_w, width):
    rpt = n // NS

    @functools.partial(
        pl.kernel,
        out_type=jax.ShapeDtypeStruct((NC, 2, n, width), jnp.float32),
        mesh=_sc_mesh(),
        compiler_params=pltpu.CompilerParams(use_tc_tiling_on_sc=False),
        scratch_types=[
            pltpu.VMEM((rows_per_w, EB), jnp.int32),
            pltpu.VMEM((rows_per_w, EB), jnp.int32),
            pltpu.VMEM((EB, width), jnp.float32),
            pltpu.VMEM_SHARED((n, width), jnp.float32),
            pltpu.VMEM_SHARED((n, width), jnp.float32),
            pltpu.SemaphoreType.DMA,
        ],
    )
    def deg_kernel(ei_hbm, ones_hbm, zeros_hbm, out_hbm, sidx, didx, ones_v,
                   table_a, table_b, sem):
        c = lax.axis_index("c")
        s = lax.axis_index("s")
        w = s * NC + c
        pltpu.sync_copy(zeros_hbm, table_a.at[pl.ds(s * rpt, rpt)])
        pltpu.sync_copy(zeros_hbm, table_b.at[pl.ds(s * rpt, rpt)])
        pltpu.sync_copy(ei_hbm.at[0, w], sidx)
        pltpu.sync_copy(ei_hbm.at[1, w], didx)
        pltpu.sync_copy(ones_hbm, ones_v)
        plsc.subcore_barrier()

        def body(j, _):
            pltpu.async_copy(ones_v, table_a.at[sidx.at[j]], sem, add=True)
            pltpu.async_copy(ones_v, table_b.at[didx.at[j]], sem, add=True)

            @pl.when(j >= 2)
            def _():
                pltpu.make_async_copy(ones_v, table_a.at[sidx.at[j]], sem).wait()
                pltpu.make_async_copy(ones_v, table_a.at[sidx.at[j]], sem).wait()

            return ()

        lax.fori_loop(0, rows_per_w, body, ())
        for _ in range(4):
            pltpu.make_async_copy(ones_v, table_a.at[sidx.at[0]], sem).wait()
        plsc.subcore_barrier()
        pltpu.sync_copy(
            table_a.at[pl.ds(s * rpt, rpt)],
            out_hbm.at[c, 0, pl.ds(s * rpt, rpt)],
        )
        pltpu.sync_copy(
            table_b.at[pl.ds(s * rpt, rpt)],
            out_hbm.at[c, 1, pl.ds(s * rpt, rpt)],
        )

    return deg_kernel


def _make_agg_kernel(n, feat, rows_per_w):
    rpt = n // NS

    @functools.partial(
        pl.kernel,
        out_type=jax.ShapeDtypeStruct((NC, n, feat), jnp.float32),
        mesh=_sc_mesh(),
        compiler_params=pltpu.CompilerParams(use_tc_tiling_on_sc=False),
        scratch_types=[
            pltpu.VMEM((rows_per_w, EB), jnp.int32),
            pltpu.VMEM((rows_per_w, EB), jnp.int32),
            pltpu.VMEM((NBUF, EB, feat), jnp.float32),
            pltpu.VMEM_SHARED((n, feat), jnp.float32),
            pltpu.SemaphoreType.DMA((NBUF,)),
            pltpu.SemaphoreType.DMA((NBUF,)),
        ],
    )
    def agg_kernel(h_hbm, ei_hbm, zeros_hbm, out_hbm, sidx, didx, rows,
                   table, gsems, ssems):
        c = lax.axis_index("c")
        s = lax.axis_index("s")
        w = s * NC + c
        pltpu.sync_copy(zeros_hbm, table.at[pl.ds(s * rpt, rpt)])
        pltpu.sync_copy(ei_hbm.at[0, w], sidx)
        pltpu.sync_copy(ei_hbm.at[1, w], didx)
        plsc.subcore_barrier()

        for k in range(NBUF - 1):
            pltpu.async_copy(h_hbm.at[sidx.at[k]], rows.at[k], gsems.at[k])

        def body(j, _):
            slot = lax.rem(j, NBUF)
            nxt = j + NBUF - 1

            @pl.when(nxt < rows_per_w)
            def _():
                nslot = lax.rem(nxt, NBUF)

                @pl.when(j >= 1)
                def _():
                    pltpu.make_async_copy(
                        rows.at[nslot], table.at[didx.at[j]],
                        ssems.at[nslot]).wait()

                pltpu.async_copy(
                    h_hbm.at[sidx.at[nxt]], rows.at[nslot], gsems.at[nslot])

            pltpu.make_async_copy(
                h_hbm.at[sidx.at[j]], rows.at[slot], gsems.at[slot]).wait()
            pltpu.async_copy(
                rows.at[slot], table.at[didx.at[j]], ssems.at[slot], add=True)
            return ()

        lax.fori_loop(0, rows_per_w, body, ())
        for k in range(NBUF):
            pltpu.make_async_copy(
                rows.at[k], table.at[didx.at[0]], ssems.at[k]).wait()
        plsc.subcore_barrier()
        pltpu.sync_copy(
            table.at[pl.ds(s * rpt, rpt)],
            out_hbm.at[c, pl.ds(s * rpt, rpt)],
        )

    return agg_kernel


def _norm_from(degp_blk):
    deg = degp_blk[0, 0, :, 0:1] + degp_blk[1, 0, :, 0:1]
    return lax.rsqrt(jnp.clip(deg, 1.0, None))


def _m1_body(x_ref, w0_ref, degp_ref, o_ref):
    h = jnp.dot(x_ref[...], w0_ref[...], preferred_element_type=jnp.float32)
    o_ref[...] = h * _norm_from(degp_ref[...])


def _m2_body(aggp_ref, degin_ref, degout_ref, b0_ref, o_ref):
    agg = aggp_ref[0] + aggp_ref[1]
    hidden = jnp.maximum(agg * _norm_from(degin_ref[...]) + b0_ref[...], 0.0)
    o_ref[...] = hidden * _norm_from(degout_ref[...])


def _m3_body(aggp_ref, degin_ref, noise_ref, w1_ref, b1_ref, w2_ref, b2_ref, o_ref):
    t = (aggp_ref[0] + aggp_ref[1]) * _norm_from(degin_ref[...])
    mean = jnp.dot(t, w1_ref[...], preferred_element_type=jnp.float32) + b1_ref[...]
    logstd = jnp.dot(t, w2_ref[...], preferred_element_type=jnp.float32) + b2_ref[...]
    o_ref[...] = noise_ref[...] * jnp.exp(logstd) + mean


def _m4_body(za_ref, zb_ref, o_ref):
    prod = lax.dot_general(
        za_ref[...], zb_ref[...], (((1,), (1,)), ((), ())),
        preferred_element_type=jnp.float32,
    )
    o_ref[...] = 0.5 * jnp.tanh(0.5 * prod) + 0.5


def kernel(x, edge_index, noise, W0, b0, W1, b1, W2, b2):
    n, d = x.shape
    e = edge_index.shape[1]
    h1 = W0.shape[1]
    h2 = W1.shape[1]
    rb = 5000
    gi = n // rb
    dw = 16
    npad = ((n + NS * 8 - 1) // (NS * 8)) * (NS * 8)

    epw = e // NW
    ei4 = edge_index.reshape(2, NW, epw // EB, EB)

    ones_b = jnp.ones((EB, dw), jnp.float32)
    zeros_deg = jnp.zeros((npad // NS, dw), jnp.float32)
    zeros_agg = jnp.zeros((npad // NS, h1), jnp.float32)

    deg_k = _make_degree_kernel(npad, epw // EB, dw)
    degp = deg_k(ei4, ones_b, zeros_deg)

    b0r = b0.reshape(1, h1)
    b1r = b1.reshape(1, h2)
    b2r = b2.reshape(1, h2)

    h_scaled = pl.pallas_call(
        _m1_body,
        grid=(gi,),
        in_specs=[
            pl.BlockSpec((rb, d), lambda i: (i, 0)),
            pl.BlockSpec((d, h1), lambda i: (0, 0)),
            pl.BlockSpec((NC, 1, rb, dw), lambda i: (0, 0, i, 0)),
        ],
        out_specs=pl.BlockSpec((rb, h1), lambda i: (i, 0)),
        out_shape=jax.ShapeDtypeStruct((n, h1), jnp.float32),
    )(x, W0, degp)

    agg_k = _make_agg_kernel(npad, h1, epw // EB)
    agg1p = agg_k(h_scaled, ei4, zeros_agg)

    hidden_scaled = pl.pallas_call(
        _m2_body,
        grid=(gi,),
        in_specs=[
            pl.BlockSpec((NC, rb, h1), lambda i: (0, i, 0)),
            pl.BlockSpec((NC, 1, rb, dw), lambda i: (0, 1, i, 0)),
            pl.BlockSpec((NC, 1, rb, dw), lambda i: (0, 0, i, 0)),
            pl.BlockSpec((1, h1), lambda i: (0, 0)),
        ],
        out_specs=pl.BlockSpec((rb, h1), lambda i: (i, 0)),
        out_shape=jax.ShapeDtypeStruct((n, h1), jnp.float32),
    )(agg1p, degp, degp, b0r)

    agg2p = agg_k(hidden_scaled, ei4, zeros_agg)

    z = pl.pallas_call(
        _m3_body,
        grid=(gi,),
        in_specs=[
            pl.BlockSpec((NC, rb, h1), lambda i: (0, i, 0)),
            pl.BlockSpec((NC, 1, rb, dw), lambda i: (0, 1, i, 0)),
            pl.BlockSpec((rb, h2), lambda i: (i, 0)),
            pl.BlockSpec((h1, h2), lambda i: (0, 0)),
            pl.BlockSpec((1, h2), lambda i: (0, 0)),
            pl.BlockSpec((h1, h2), lambda i: (0, 0)),
            pl.BlockSpec((1, h2), lambda i: (0, 0)),
        ],
        out_specs=pl.BlockSpec((rb, h2), lambda i: (i, 0)),
        out_shape=jax.ShapeDtypeStruct((n, h2), jnp.float32),
    )(agg2p, degp, noise, W1, b1r, W2, b2r)

    rdec = 400
    a_pred = pl.pallas_call(
        _m4_body,
        grid=(n // rdec,),
        in_specs=[
            pl.BlockSpec((rdec, h2), lambda i: (i, 0)),
            pl.BlockSpec((n, h2), lambda i: (0, 0)),
        ],
        out_specs=pl.BlockSpec((rdec, n), lambda i: (i, 0)),
        out_shape=jax.ShapeDtypeStruct((n, n), jnp.float32),
    )(z, z)

    return (z, a_pred)
